# baseline (speedup 1.0000x reference)
import functools

import jax
import jax.numpy as jnp
from jax import lax
from jax.experimental import pallas as pl
from jax.experimental.pallas import tpu as pltpu
from jax.experimental.pallas import tpu_sc as plsc

_COMMITMENT_COST = 0.25
_BM = 2048
_NTILE = 4096

_SC_CORES = 2
_SC_SUBCORES = 16
_NW = _SC_CORES * _SC_SUBCORES
_ICH = 128


def _dist_argmin_body(z_ref, z2_ref, w_ref, w2_ref, iota_ref, idx_ref):
    n_total = w_ref.shape[0]
    zb = (2.0 * z_ref[...]).astype(jnp.bfloat16)
    z2 = z2_ref[...]
    run_min = None
    run_idx = None
    for c in range(0, n_total, _NTILE):
        wblk = w_ref[c:c + _NTILE, :]
        w2 = w2_ref[:, c:c + _NTILE]
        mm = lax.dot_general(zb, wblk, (((1,), (1,)), ((), ())),
                             preferred_element_type=jnp.float32)
        d = (z2 - mm) + w2
        mn = jnp.min(d, axis=1, keepdims=True)
        col = iota_ref[:, c:c + _NTILE]
        cidx = jnp.min(jnp.where(d == mn, col, jnp.float32(jnp.inf)),
                       axis=1, keepdims=True)
        if run_min is None:
            run_idx = cidx
        else:
            upd = mn < run_min
            run_idx = jnp.where(upd, cidx, run_idx)
            mn = jnp.where(upd, mn, run_min)
        run_min = mn.astype(jnp.bfloat16).astype(jnp.float32)
    idx_ref[...] = run_idx.astype(jnp.int32)


def _tc_dist_argmin(flat_z, z2, W, w2, iota_row):
    m, d = flat_z.shape
    n = W.shape[0]
    grid = (m // _BM,)
    return pl.pallas_call(
        _dist_argmin_body,
        grid=grid,
        in_specs=[
            pl.BlockSpec((_BM, d), lambda i: (i, 0)),
            pl.BlockSpec((_BM, 1), lambda i: (i, 0)),
            pl.BlockSpec((n, d), lambda i: (0, 0)),
            pl.BlockSpec((1, n), lambda i: (0, 0)),
            pl.BlockSpec((1, n), lambda i: (0, 0)),
        ],
        out_specs=pl.BlockSpec((_BM, 1), lambda i: (i, 0)),
        out_shape=jax.ShapeDtypeStruct((m, 1), jnp.int32),
    )(flat_z, z2, W, w2, iota_row)


def _sc_gather_st(table, idx_flat, flat_z):
    m = idx_flat.shape[0]
    d = table.shape[1]
    rpw = m // _NW
    nch = rpw // _ICH
    nvec = d // 16
    idx3 = idx_flat.reshape(_NW, nch, _ICH)
    mesh = plsc.VectorSubcoreMesh(core_axis_name="c", subcore_axis_name="s")

    @functools.partial(
        pl.kernel, mesh=mesh,
        out_type=[jax.ShapeDtypeStruct((m, d), jnp.float32),
                  jax.ShapeDtypeStruct((_NW, 16), jnp.float32)],
        compiler_params=pltpu.CompilerParams(use_tc_tiling_on_sc=False),
        scratch_types=[
            pltpu.VMEM((nch, _ICH), jnp.int32),
            pltpu.VMEM((rpw, d), jnp.float32),
            pltpu.VMEM((rpw, d), jnp.float32),
            pltpu.VMEM((rpw, d), jnp.float32),
            pltpu.VMEM((16,), jnp.float32),
            pltpu.SemaphoreType.DMA,
            pltpu.SemaphoreType.DMA,
        ],
    )
    def gk(table_hbm, idx_hbm, z_hbm, out_hbm, part_hbm,
           idx_v, rows_v, z_v, o_v, part_v, sem, zsem):
        wid = lax.axis_index("s") * _SC_CORES + lax.axis_index("c")
        base = wid * rpw
        zcp = pltpu.async_copy(z_hbm.at[pl.ds(base, rpw)], z_v, zsem)
        pltpu.sync_copy(idx_hbm.at[wid], idx_v)
        for cc in range(nch):
            pltpu.async_copy(table_hbm.at[idx_v.at[cc]],
                             rows_v.at[pl.ds(cc * _ICH, _ICH)], sem).wait()
        zcp.wait()

        def row_body(i, acc):
            for h in range(nvec):
                sl = pl.ds(h * 16, 16)
                zq = rows_v[i, sl]
                zz = z_v[i, sl]
                t = zq - zz
                o_v[i, sl] = zz + t
                acc = acc + t * t
            return acc

        acc = lax.fori_loop(0, rpw, row_body,
                            jnp.zeros((16,), jnp.float32))
        pltpu.sync_copy(o_v, out_hbm.at[pl.ds(base, rpw)])
        part_v[...] = acc
        pltpu.sync_copy(part_v, part_hbm.at[wid])

    return gk(table, idx3, flat_z)


def kernel(z, W):
    B, T, D = z.shape
    N = W.shape[0]
    flat_z = z.reshape(-1, D)
    z2 = jnp.sum(flat_z ** 2, axis=-1, keepdims=True)
    w2 = jnp.sum(W ** 2, axis=-1).reshape(1, N)
    iota_row = jnp.arange(N, dtype=jnp.float32).reshape(1, N)
    idx2d = _tc_dist_argmin(flat_z, z2, W, w2, iota_row)
    indices_flat = idx2d.reshape(-1)
    z_q_st_flat, loss_parts = _sc_gather_st(W, indices_flat, flat_z)
    z_q_st = z_q_st_flat.reshape(B, T, D)
    loss = _COMMITMENT_COST * (jnp.sum(loss_parts) / (B * T * D))
    indices = indices_flat.reshape(B, T)
    return (z_q_st, loss, indices)
